# Initial kernel scaffold; baseline (speedup 1.0000x reference)
import jax
import jax.numpy as jnp
from jax.experimental import pallas as pl


def kernel(Z, edge_index, edge_type, dist, node_emb, edge_emb, Wn1, Wn2, We1, We2, Wc, Wr1, Wr2):
    raise NotImplementedError("write your pallas kernel here")


# speedup vs baseline: 2.1957x; 2.1957x over previous
import functools

import jax
import jax.numpy as jnp
from jax import lax
from jax.experimental import pallas as pl
from jax.experimental.pallas import tpu as pltpu
from jax.experimental.pallas import tpu_sc as plsc

_N = 10000
_E = 160000
_DN = 128
_DE = 128
_NC = 64
_DEF = _DE + _NC
_NCONV = 3
_NODE_DICT = 20
_EDGE_DICT = 400
_CLOW, _CHIGH = 0.0, 30.0
_GAP = (_CHIGH - _CLOW) / (_NC - 1)
_FH = _DEF // 2

_NCORE = 2
_NSUB = 16
_CH = 128
_NCHUNK = 79
_EPW = _CH * _NCHUNK
_EP = _EPW * _NSUB
_NPS = _N // _NSUB
_ZR = 125
_NLANE = 16
_NVEC = _FH // _NLANE

_TE = 2048
_TN = 2000


def _node_init_body(z_ref, emb_ref, wn1_ref, wn2_ref, h0_ref, hn0_ref):
    z = z_ref[0, 0, :]
    oh = (z[:, None] == lax.broadcasted_iota(jnp.int32, (_TN, _NODE_DICT), 1))
    h0 = oh.astype(jnp.float32) @ emb_ref[...]
    h0_ref[...] = h0
    hn = jnp.maximum(h0 @ wn1_ref[...], 0.0) @ wn2_ref[...]
    hn0_ref[0, :, :] = hn[:, :_FH]
    hn0_ref[1, :, :] = hn[:, _FH:]


def _node_init(Zr, node_emb, Wn1_0, Wn2_0):
    return pl.pallas_call(
        _node_init_body,
        grid=(_N // _TN,),
        in_specs=[
            pl.BlockSpec((1, 1, _TN), lambda t: (t, 0, 0)),
            pl.BlockSpec((_NODE_DICT, _DN), lambda t: (0, 0)),
            pl.BlockSpec((_DN, _DN), lambda t: (0, 0)),
            pl.BlockSpec((_DN, _DEF), lambda t: (0, 0)),
        ],
        out_specs=[
            pl.BlockSpec((_TN, _DN), lambda t: (t, 0)),
            pl.BlockSpec((_NCORE, _TN, _FH), lambda t: (0, t, 0)),
        ],
        out_shape=[
            jax.ShapeDtypeStruct((_N, _DN), jnp.float32),
            jax.ShapeDtypeStruct((_NCORE, _N, _FH), jnp.float32),
        ],
    )(Zr, node_emb, Wn1_0, Wn2_0)


def _he_body(et_ref, dist_ref, emb_ref, we1_ref, we2_ref,
             he0_ref, he1_ref, he2_ref):
    t = pl.program_id(0)
    et = et_ref[0, 0, :]
    oh = (et[:, None] == lax.broadcasted_iota(jnp.int32, (_TE, _EDGE_DICT), 1))
    eh1 = oh.astype(jnp.float32) @ emb_ref[...]
    d = dist_ref[0, 0, :]
    centers = _CLOW + _GAP * lax.broadcasted_iota(
        jnp.int32, (1, _NC), 1).astype(jnp.float32)
    rbf = jnp.exp(-((d[:, None] - centers) ** 2) / (_GAP * _GAP))
    eh = jnp.concatenate([eh1, rbf], axis=1)
    eid = t * _TE + lax.broadcasted_iota(jnp.int32, (_TE, 1), 0)
    mask = eid < _E
    for i, out_ref in enumerate((he0_ref, he1_ref, he2_ref)):
        he = jnp.maximum(eh @ we1_ref[i], 0.0) @ we2_ref[i]
        he = jnp.where(mask, he, 0.0)
        out_ref[0, :, :] = he[:, :_FH]
        out_ref[1, :, :] = he[:, _FH:]


def _he_precompute(et_p, dist_p, edge_emb, We1, We2):
    edge_out = jax.ShapeDtypeStruct((_NCORE, _EP, _FH), jnp.float32)
    return pl.pallas_call(
        _he_body,
        grid=(_EP // _TE,),
        in_specs=[
            pl.BlockSpec((1, 1, _TE), lambda t: (t, 0, 0)),
            pl.BlockSpec((1, 1, _TE), lambda t: (t, 0, 0)),
            pl.BlockSpec((_EDGE_DICT, _DE), lambda t: (0, 0)),
            pl.BlockSpec((_NCONV, _DEF, _DEF), lambda t: (0, 0, 0)),
            pl.BlockSpec((_NCONV, _DEF, _DEF), lambda t: (0, 0, 0)),
        ],
        out_specs=[pl.BlockSpec((_NCORE, _TE, _FH), lambda t: (0, t, 0))] * 3,
        out_shape=[edge_out] * 3,
    )(et_p, dist_p, edge_emb, We1, We2)


_SC_MESH = plsc.VectorSubcoreMesh(core_axis_name="c", subcore_axis_name="s")


@functools.partial(
    pl.kernel,
    out_type=jax.ShapeDtypeStruct((_NCORE, _N, _FH), jnp.float32),
    mesh=_SC_MESH,
    scratch_types=[
        pltpu.VMEM((_CH,), jnp.int32),
        pltpu.VMEM((_CH,), jnp.int32),
        pltpu.VMEM((_CH, _FH), jnp.float32),
        pltpu.VMEM((_CH, _FH), jnp.float32),
        pltpu.VMEM((_ZR, _FH), jnp.float32),
        pltpu.VMEM_SHARED((_N, _FH), jnp.float32),
        pltpu.SemaphoreType.DMA,
    ],
    compiler_params=pltpu.CompilerParams(use_tc_tiling_on_sc=False),
)
def _sc_message_pass(hn_hbm, he_hbm, src_hbm, dst_hbm, out_hbm,
                     src_v, dst_v, g_v, m_v, zero_v, agg_sh, sem):
    c = lax.axis_index("c")
    s = lax.axis_index("s")

    def _zfill(e, carry):
        for j in range(_NVEC):
            zero_v[e, pl.ds(j * _NLANE, _NLANE)] = jnp.zeros((_NLANE,),
                                                             jnp.float32)
        return carry
    lax.fori_loop(0, _ZR, _zfill, 0)
    for k in range(_NPS // _ZR):
        pltpu.sync_copy(zero_v, agg_sh.at[pl.ds(s * _NPS + k * _ZR, _ZR), :])
    plsc.subcore_barrier()

    base = s * _EPW

    def _chunk(k, carry):
        b = base + k * _CH
        pltpu.sync_copy(src_hbm.at[pl.ds(b, _CH)], src_v)
        pltpu.sync_copy(dst_hbm.at[pl.ds(b, _CH)], dst_v)
        pltpu.async_copy(hn_hbm.at[c].at[src_v], g_v, sem).wait()
        pltpu.sync_copy(he_hbm.at[c, pl.ds(b, _CH), :], m_v)

        def _mul(e, cc):
            for j in range(_NVEC):
                sl = pl.ds(j * _NLANE, _NLANE)
                m_v[e, sl] = m_v[e, sl] * g_v[e, sl]
            return cc
        lax.fori_loop(0, _CH, _mul, 0)
        pltpu.sync_copy(m_v, agg_sh.at[dst_v], add=True)
        return carry
    lax.fori_loop(0, _NCHUNK, _chunk, 0)
    plsc.subcore_barrier()

    pltpu.sync_copy(agg_sh.at[pl.ds(s * _NPS, _NPS), :],
                    out_hbm.at[c, pl.ds(s * _NPS, _NPS), :])


def _upd_hn_body(agg_ref, h_ref, wc_ref, wn1_ref, wn2_ref, h_out, hn_out):
    a = jnp.concatenate([agg_ref[0], agg_ref[1]], axis=1)
    h = h_ref[...] + jnp.tanh(a @ wc_ref[...])
    h_out[...] = h
    hn = jnp.maximum(h @ wn1_ref[...], 0.0) @ wn2_ref[...]
    hn_out[0, :, :] = hn[:, :_FH]
    hn_out[1, :, :] = hn[:, _FH:]


def _upd_hn(aggp, h, Wc_i, Wn1_n, Wn2_n):
    return pl.pallas_call(
        _upd_hn_body,
        grid=(_N // _TN,),
        in_specs=[
            pl.BlockSpec((_NCORE, _TN, _FH), lambda t: (0, t, 0)),
            pl.BlockSpec((_TN, _DN), lambda t: (t, 0)),
            pl.BlockSpec((_DEF, _DN), lambda t: (0, 0)),
            pl.BlockSpec((_DN, _DN), lambda t: (0, 0)),
            pl.BlockSpec((_DN, _DEF), lambda t: (0, 0)),
        ],
        out_specs=[
            pl.BlockSpec((_TN, _DN), lambda t: (t, 0)),
            pl.BlockSpec((_NCORE, _TN, _FH), lambda t: (0, t, 0)),
        ],
        out_shape=[
            jax.ShapeDtypeStruct((_N, _DN), jnp.float32),
            jax.ShapeDtypeStruct((_NCORE, _N, _FH), jnp.float32),
        ],
    )(aggp, h, Wc_i, Wn1_n, Wn2_n)


def _upd_read_body(agg_ref, h_ref, wc_ref, wr1_ref, wr2_ref, out_ref):
    a = jnp.concatenate([agg_ref[0], agg_ref[1]], axis=1)
    h = h_ref[...] + jnp.tanh(a @ wc_ref[...])
    r = jnp.maximum(h @ wr1_ref[...], 0.0)
    row = jnp.sum(r * wr2_ref[...], axis=0, keepdims=True)

    @pl.when(pl.program_id(0) == 0)
    def _():
        out_ref[...] = jnp.zeros((1, _DN), jnp.float32)
    out_ref[...] += row

    @pl.when(pl.program_id(0) == _N // _TN - 1)
    def _():
        out_ref[...] = jnp.zeros((1, _DN), jnp.float32) + jnp.sum(out_ref[...])


def _upd_read(aggp, h, Wc_i, Wr1, Wr2_row):
    return pl.pallas_call(
        _upd_read_body,
        grid=(_N // _TN,),
        in_specs=[
            pl.BlockSpec((_NCORE, _TN, _FH), lambda t: (0, t, 0)),
            pl.BlockSpec((_TN, _DN), lambda t: (t, 0)),
            pl.BlockSpec((_DEF, _DN), lambda t: (0, 0)),
            pl.BlockSpec((_DN, _DN), lambda t: (0, 0)),
            pl.BlockSpec((1, _DN), lambda t: (0, 0)),
        ],
        out_specs=pl.BlockSpec((1, _DN), lambda t: (0, 0)),
        out_shape=jax.ShapeDtypeStruct((1, _DN), jnp.float32),
    )(aggp, h, Wc_i, Wr1, Wr2_row)


def kernel(Z, edge_index, edge_type, dist, node_emb, edge_emb,
           Wn1, Wn2, We1, We2, Wc, Wr1, Wr2):
    pad = _EP - _E
    et_p = jnp.pad(edge_type, (0, pad)).reshape(_EP // _TE, 1, _TE)
    dist_p = jnp.pad(dist, (0, pad)).reshape(_EP // _TE, 1, _TE)
    src_p = jnp.pad(edge_index[0], (0, pad))
    dst_p = jnp.pad(edge_index[1], (0, pad))
    Zr = Z.reshape(_N // _TN, 1, _TN)

    h, hn = _node_init(Zr, node_emb, Wn1[0], Wn2[0])
    hes = _he_precompute(et_p, dist_p, edge_emb, We1, We2)

    for i in range(_NCONV):
        aggp = _sc_message_pass(hn, hes[i], src_p, dst_p)
        if i + 1 < _NCONV:
            h, hn = _upd_hn(aggp, h, Wc[i], Wn1[i + 1], Wn2[i + 1])
        else:
            out = _upd_read(aggp, h, Wc[i], Wr1, Wr2.reshape(1, _DN))
    return out[0, :1]
